# baseline (speedup 1.0000x reference)
import jax
import jax.numpy as jnp
from jax.experimental import pallas as pl
from jax.experimental.pallas import tpu as pltpu

E = 16
D = 1024
H = 4096
R = 4
BATCH = 32

NC1 = 16
NC2 = 16
CH1 = H // NC1
CH2 = D // NC2


def _gate_kernel(x_ref, wg_ref, bg_ref, w_ref, idx_ref):
    x = x_ref[...]
    logits = jax.lax.dot_general(
        x, wg_ref[...], (((1,), (1,)), ((), ())),
        preferred_element_type=jnp.float32) + bg_ref[...]
    m = jnp.max(logits, axis=-1, keepdims=True)
    ex = jnp.exp(logits - m)
    probs = ex / jnp.sum(ex, axis=-1, keepdims=True)
    w_ref[...] = jnp.max(probs, axis=-1, keepdims=True)
    idx_ref[...] = jnp.argmax(probs, axis=-1, keepdims=True).astype(jnp.int32)


def _dot_nt(a, b):
    return jax.lax.dot_general(a, b, (((1,), (1,)), ((), ())),
                               preferred_element_type=jnp.float32)


def _dot_nn(a, b):
    return jax.lax.dot_general(a, b, (((1,), (0,)), ((), ())),
                               preferred_element_type=jnp.float32)


def _ffn_kernel(idx_ref, x_ref, w1_hbm, b1_ref, a1_ref, bb1_ref,
                w2_hbm, b2_ref, a2_ref, bb2_ref, w_ref, out_ref,
                w1v, w2v, h_ref, sem1, sem2):
    e = idx_ref[0, 0]
    cps1 = [
        pltpu.make_async_copy(
            w1_hbm.at[e, pl.ds(c * CH1, CH1), :],
            w1v.at[pl.ds(c * CH1, CH1), :], sem1.at[c])
        for c in range(NC1)
    ]
    cps2 = [
        pltpu.make_async_copy(
            w2_hbm.at[e, pl.ds(c * CH2, CH2), :],
            w2v.at[pl.ds(c * CH2, CH2), :], sem2.at[c])
        for c in range(NC2)
    ]
    for cp in cps1:
        cp.start()
    for cp in cps2:
        cp.start()

    x = x_ref[...]
    t1 = _dot_nt(x, bb1_ref[0])
    for c in range(NC1):
        cps1[c].wait()
        h = _dot_nt(x, w1v[c * CH1:(c + 1) * CH1, :])
        h = h + _dot_nn(t1, a1_ref[0, :, c * CH1:(c + 1) * CH1])
        h = h + b1_ref[0, :, c * CH1:(c + 1) * CH1]
        h_ref[:, c * CH1:(c + 1) * CH1] = jnp.maximum(h, 0.0)

    hfull = h_ref[...]
    t2 = _dot_nt(hfull, bb2_ref[0])
    for c in range(NC2):
        cps2[c].wait()
        p = _dot_nt(hfull, w2v[c * CH2:(c + 1) * CH2, :])
        p = p + _dot_nn(t2, a2_ref[0, :, c * CH2:(c + 1) * CH2])
        p = p + b2_ref[0, :, c * CH2:(c + 1) * CH2]
        out_ref[:, c * CH2:(c + 1) * CH2] = p * w_ref[...]


@jax.jit
def kernel(x, Wg, bg, W1, b1, A1, B1, W2, b2, A2, B2):
    topw, topi = pl.pallas_call(
        _gate_kernel,
        out_shape=(
            jax.ShapeDtypeStruct((BATCH, 1), jnp.float32),
            jax.ShapeDtypeStruct((BATCH, 1), jnp.int32),
        ),
    )(x, Wg, bg.reshape(1, E))

    grid_spec = pltpu.PrefetchScalarGridSpec(
        num_scalar_prefetch=1,
        grid=(1,),
        in_specs=[
            pl.BlockSpec((BATCH, D), lambda i, e: (0, 0)),
            pl.BlockSpec(memory_space=pltpu.MemorySpace.HBM),
            pl.BlockSpec((1, 1, H), lambda i, e: (e[0, 0], 0, 0)),
            pl.BlockSpec((1, R, H), lambda i, e: (e[0, 0], 0, 0)),
            pl.BlockSpec((1, R, D), lambda i, e: (e[0, 0], 0, 0)),
            pl.BlockSpec(memory_space=pltpu.MemorySpace.HBM),
            pl.BlockSpec((1, 1, D), lambda i, e: (e[0, 0], 0, 0)),
            pl.BlockSpec((1, R, D), lambda i, e: (e[0, 0], 0, 0)),
            pl.BlockSpec((1, R, H), lambda i, e: (e[0, 0], 0, 0)),
            pl.BlockSpec((BATCH, 1), lambda i, e: (0, 0)),
        ],
        out_specs=pl.BlockSpec((BATCH, D), lambda i, e: (0, 0)),
        scratch_shapes=[
            pltpu.VMEM((H, D), jnp.float32),
            pltpu.VMEM((D, H), jnp.float32),
            pltpu.VMEM((BATCH, H), jnp.float32),
            pltpu.SemaphoreType.DMA((NC1,)),
            pltpu.SemaphoreType.DMA((NC2,)),
        ],
    )
    out = pl.pallas_call(
        _ffn_kernel,
        grid_spec=grid_spec,
        out_shape=jax.ShapeDtypeStruct((BATCH, D), jnp.float32),
    )(topi, x, W1, b1.reshape(E, 1, H), A1.transpose(0, 2, 1), B1, W2,
      b2.reshape(E, 1, D), A2.transpose(0, 2, 1), B2, topw)
    return (out, topi)


# speedup vs baseline: 1.0317x; 1.0317x over previous
import jax
import jax.numpy as jnp
from jax.experimental import pallas as pl
from jax.experimental.pallas import tpu as pltpu

E = 16
D = 1024
H = 4096
R = 4
BATCH = 32

NC1 = 4
NC2 = 4
CH1 = H // NC1
CH2 = D // NC2


def _gate_kernel(x_ref, wg_ref, bg_ref, w_ref, idx_ref):
    x = x_ref[...]
    logits = jax.lax.dot_general(
        x, wg_ref[...], (((1,), (1,)), ((), ())),
        preferred_element_type=jnp.float32) + bg_ref[...]
    m = jnp.max(logits, axis=-1, keepdims=True)
    ex = jnp.exp(logits - m)
    probs = ex / jnp.sum(ex, axis=-1, keepdims=True)
    w_ref[...] = jnp.max(probs, axis=-1, keepdims=True)
    idx_ref[...] = jnp.argmax(probs, axis=-1, keepdims=True).astype(jnp.int32)


def _dot_nt(a, b):
    return jax.lax.dot_general(a, b, (((1,), (1,)), ((), ())),
                               preferred_element_type=jnp.float32)


def _dot_nn(a, b):
    return jax.lax.dot_general(a, b, (((1,), (0,)), ((), ())),
                               preferred_element_type=jnp.float32)


def _ffn_kernel(idx_ref, x_ref, w1_hbm, b1_ref, a1_ref, bb1_ref,
                w2_hbm, b2_ref, a2_ref, bb2_ref, w_ref, out_ref,
                w1v, w2v, h_ref, sem1, sem2):
    e = idx_ref[0, 0]
    cps1 = [
        pltpu.make_async_copy(
            w1_hbm.at[e, pl.ds(c * CH1, CH1), :],
            w1v.at[pl.ds(c * CH1, CH1), :], sem1.at[c])
        for c in range(NC1)
    ]
    cps2 = [
        pltpu.make_async_copy(
            w2_hbm.at[e, pl.ds(c * CH2, CH2), :],
            w2v.at[pl.ds(c * CH2, CH2), :], sem2.at[c])
        for c in range(NC2)
    ]
    for cp in cps1:
        cp.start()
    for cp in cps2:
        cp.start()

    x = x_ref[...]
    t1 = _dot_nt(x, bb1_ref[0])
    for c in range(NC1):
        cps1[c].wait()
        h = _dot_nt(x, w1v[c * CH1:(c + 1) * CH1, :])
        h = h + _dot_nn(t1, a1_ref[0, :, c * CH1:(c + 1) * CH1])
        h = h + b1_ref[0, :, c * CH1:(c + 1) * CH1]
        h_ref[:, c * CH1:(c + 1) * CH1] = jnp.maximum(h, 0.0)

    hfull = h_ref[...]
    t2 = _dot_nt(hfull, bb2_ref[0])
    for c in range(NC2):
        cps2[c].wait()
        p = _dot_nt(hfull, w2v[c * CH2:(c + 1) * CH2, :])
        p = p + _dot_nn(t2, a2_ref[0, :, c * CH2:(c + 1) * CH2])
        p = p + b2_ref[0, :, c * CH2:(c + 1) * CH2]
        out_ref[:, c * CH2:(c + 1) * CH2] = p * w_ref[...]


@jax.jit
def kernel(x, Wg, bg, W1, b1, A1, B1, W2, b2, A2, B2):
    topw, topi = pl.pallas_call(
        _gate_kernel,
        out_shape=(
            jax.ShapeDtypeStruct((BATCH, 1), jnp.float32),
            jax.ShapeDtypeStruct((BATCH, 1), jnp.int32),
        ),
    )(x, Wg, bg.reshape(1, E))

    grid_spec = pltpu.PrefetchScalarGridSpec(
        num_scalar_prefetch=1,
        grid=(1,),
        in_specs=[
            pl.BlockSpec((BATCH, D), lambda i, e: (0, 0)),
            pl.BlockSpec(memory_space=pltpu.MemorySpace.HBM),
            pl.BlockSpec((1, 1, H), lambda i, e: (e[0, 0], 0, 0)),
            pl.BlockSpec((1, R, H), lambda i, e: (e[0, 0], 0, 0)),
            pl.BlockSpec((1, R, D), lambda i, e: (e[0, 0], 0, 0)),
            pl.BlockSpec(memory_space=pltpu.MemorySpace.HBM),
            pl.BlockSpec((1, 1, D), lambda i, e: (e[0, 0], 0, 0)),
            pl.BlockSpec((1, R, D), lambda i, e: (e[0, 0], 0, 0)),
            pl.BlockSpec((1, R, H), lambda i, e: (e[0, 0], 0, 0)),
            pl.BlockSpec((BATCH, 1), lambda i, e: (0, 0)),
        ],
        out_specs=pl.BlockSpec((BATCH, D), lambda i, e: (0, 0)),
        scratch_shapes=[
            pltpu.VMEM((H, D), jnp.float32),
            pltpu.VMEM((D, H), jnp.float32),
            pltpu.VMEM((BATCH, H), jnp.float32),
            pltpu.SemaphoreType.DMA((NC1,)),
            pltpu.SemaphoreType.DMA((NC2,)),
        ],
    )
    out = pl.pallas_call(
        _ffn_kernel,
        grid_spec=grid_spec,
        out_shape=jax.ShapeDtypeStruct((BATCH, D), jnp.float32),
    )(topi, x, W1, b1.reshape(E, 1, H), A1.transpose(0, 2, 1), B1, W2,
      b2.reshape(E, 1, D), A2.transpose(0, 2, 1), B2, topw)
    return (out, topi)


# speedup vs baseline: 1.0359x; 1.0041x over previous
import jax
import jax.numpy as jnp
from jax.experimental import pallas as pl
from jax.experimental.pallas import tpu as pltpu

E = 16
D = 1024
H = 4096
R = 4
BATCH = 32

NC1 = 8
NC2 = 8
CH1 = H // NC1
CH2 = D // NC2


def _dot_nt(a, b):
    return jax.lax.dot_general(a, b, (((1,), (1,)), ((), ())),
                               preferred_element_type=jnp.float32)


def _dot_nn(a, b):
    return jax.lax.dot_general(a, b, (((1,), (0,)), ((), ())),
                               preferred_element_type=jnp.float32)


def _moe_kernel(x_ref, wg_ref, bg_ref, w1_hbm, b1_hbm, a1_hbm, bb1_hbm,
                w2_hbm, b2_hbm, a2_hbm, bb2_hbm, out_ref, topi_ref,
                w1v, w2v, h_ref, b1v, a1v, bb1v, b2v, a2v, bb2v,
                idx_sm, sem1, sem2, sems, semi):
    x = x_ref[...]
    logits = _dot_nt(x, wg_ref[...]) + bg_ref[...]
    m = jnp.max(logits, axis=-1, keepdims=True)
    ex = jnp.exp(logits - m)
    probs = ex / jnp.sum(ex, axis=-1, keepdims=True)
    topw = jnp.max(probs, axis=-1, keepdims=True)
    topi_ref[...] = jnp.argmax(probs, axis=-1, keepdims=True).astype(jnp.int32)

    idx_cp = pltpu.make_async_copy(topi_ref.at[0:1, :], idx_sm, semi)
    idx_cp.start()
    idx_cp.wait()
    e = idx_sm[0, 0]

    cps1 = [
        pltpu.make_async_copy(
            w1_hbm.at[e, pl.ds(c * CH1, CH1), :],
            w1v.at[pl.ds(c * CH1, CH1), :], sem1.at[c])
        for c in range(NC1)
    ]
    cps2 = [
        pltpu.make_async_copy(
            w2_hbm.at[e, pl.ds(c * CH2, CH2), :],
            w2v.at[pl.ds(c * CH2, CH2), :], sem2.at[c])
        for c in range(NC2)
    ]
    small = [
        pltpu.make_async_copy(b1_hbm.at[e], b1v, sems.at[0]),
        pltpu.make_async_copy(a1_hbm.at[e], a1v, sems.at[1]),
        pltpu.make_async_copy(bb1_hbm.at[e], bb1v, sems.at[2]),
        pltpu.make_async_copy(b2_hbm.at[e], b2v, sems.at[3]),
        pltpu.make_async_copy(a2_hbm.at[e], a2v, sems.at[4]),
        pltpu.make_async_copy(bb2_hbm.at[e], bb2v, sems.at[5]),
    ]
    for cp in small:
        cp.start()
    for cp in cps1:
        cp.start()
    for cp in cps2:
        cp.start()
    for cp in small:
        cp.wait()

    t1 = _dot_nt(x, bb1v[...])
    for c in range(NC1):
        cps1[c].wait()
        h = _dot_nt(x, w1v[c * CH1:(c + 1) * CH1, :])
        h = h + _dot_nn(t1, a1v[:, c * CH1:(c + 1) * CH1])
        h = h + b1v[:, c * CH1:(c + 1) * CH1]
        h_ref[:, c * CH1:(c + 1) * CH1] = jnp.maximum(h, 0.0)

    hfull = h_ref[...]
    t2 = _dot_nt(hfull, bb2v[...])
    for c in range(NC2):
        cps2[c].wait()
        p = _dot_nt(hfull, w2v[c * CH2:(c + 1) * CH2, :])
        p = p + _dot_nn(t2, a2v[:, c * CH2:(c + 1) * CH2])
        p = p + b2v[:, c * CH2:(c + 1) * CH2]
        out_ref[:, c * CH2:(c + 1) * CH2] = p * topw


@jax.jit
def kernel(x, Wg, bg, W1, b1, A1, B1, W2, b2, A2, B2):
    hbm = pl.BlockSpec(memory_space=pltpu.MemorySpace.HBM)
    out, topi = pl.pallas_call(
        _moe_kernel,
        in_specs=[
            pl.BlockSpec((BATCH, D), lambda: (0, 0)),
            pl.BlockSpec((E, D), lambda: (0, 0)),
            pl.BlockSpec((1, E), lambda: (0, 0)),
            hbm, hbm, hbm, hbm,
            hbm, hbm, hbm, hbm,
        ],
        out_specs=(
            pl.BlockSpec((BATCH, D), lambda: (0, 0)),
            pl.BlockSpec((BATCH, 1), lambda: (0, 0)),
        ),
        out_shape=(
            jax.ShapeDtypeStruct((BATCH, D), jnp.float32),
            jax.ShapeDtypeStruct((BATCH, 1), jnp.int32),
        ),
        scratch_shapes=[
            pltpu.VMEM((H, D), jnp.float32),
            pltpu.VMEM((D, H), jnp.float32),
            pltpu.VMEM((BATCH, H), jnp.float32),
            pltpu.VMEM((1, H), jnp.float32),
            pltpu.VMEM((R, H), jnp.float32),
            pltpu.VMEM((R, D), jnp.float32),
            pltpu.VMEM((1, D), jnp.float32),
            pltpu.VMEM((R, D), jnp.float32),
            pltpu.VMEM((R, H), jnp.float32),
            pltpu.SMEM((1, 1), jnp.int32),
            pltpu.SemaphoreType.DMA((NC1,)),
            pltpu.SemaphoreType.DMA((NC2,)),
            pltpu.SemaphoreType.DMA((6,)),
            pltpu.SemaphoreType.DMA,
        ],
    )(x, Wg, bg.reshape(1, E), W1, b1.reshape(E, 1, H),
      A1.transpose(0, 2, 1), B1, W2, b2.reshape(E, 1, D),
      A2.transpose(0, 2, 1), B2)
    return (out, topi)
